# baseline (speedup 1.0000x reference)
import jax
import jax.numpy as jnp
from jax import lax
from jax.experimental import pallas as pl
from jax.experimental.pallas import tpu as pltpu

D = 768
S = 2048
H = 12
DH = 64
DFF = 3072
NR = 8
RB = 128
UB = 8
F32 = jnp.float32


def _dot(a, b, dims=None):
    if dims is None:
        return lax.dot(a, b, preferred_element_type=F32)
    return lax.dot_general(a, b, (dims, ((), ())), preferred_element_type=F32)


def _ln_rows(x, g, b):
    m = jnp.mean(x, axis=-1, keepdims=True)
    xc = x - m
    v = jnp.mean(xc * xc, axis=-1, keepdims=True)
    return xc * lax.rsqrt(v + 1e-5) * g + b


def _ln_qkv_body(x_ref, g_ref, b_ref, wq_ref, bq_ref, wk_ref, bk_ref,
                 wv_ref, bv_ref, q_ref, k_ref, v_ref):
    xn = _ln_rows(x_ref[...], g_ref[...], b_ref[...])
    q_ref[...] = _dot(xn, wq_ref[...]) + bq_ref[...]
    k_ref[...] = _dot(xn, wk_ref[...]) + bk_ref[...]
    v_ref[...] = _dot(xn, wv_ref[...]) + bv_ref[...]


def _block_body(x_ref, g_ref, b_ref, wq_ref, bq_ref, wk_ref, bk_ref,
                wv_ref, bv_ref, wo_ref, bo_ref, w1_ref, b1_ref, w2_ref,
                b2_ref, n_ref, cnt_ref, k_s, v_s):
    i = pl.program_id(0)
    nb = S // RB

    @pl.when(i < nb)
    def _stage_qkv():
        xn = _ln_rows(x_ref[...], g_ref[...], b_ref[...])
        r = pl.ds(i * RB, RB)
        k_s[r, :] = _dot(xn, wk_ref[...]) + bk_ref[...]
        v_s[r, :] = _dot(xn, wv_ref[...]) + bv_ref[...]

    @pl.when(i >= nb)
    def _stage_attn_ffn():
        j = i - nb
        xn = _ln_rows(x_ref[...], g_ref[...], b_ref[...])
        q = _dot(xn, wq_ref[...]) + bq_ref[...]
        am = jnp.zeros((RB, S), F32)
        y = jnp.broadcast_to(bo_ref[...], (RB, D))
        for h in range(H):
            sl = slice(h * DH, (h + 1) * DH)
            s = _dot(q[:, sl], k_s[:, sl], ((1,), (1,))) * 0.125
            m = jnp.max(s, axis=-1, keepdims=True)
            e = jnp.exp(s - m)
            p = e / jnp.sum(e, axis=-1, keepdims=True)
            am = am + p
            y = y + _dot(_dot(p, v_s[:, sl]), wo_ref[sl, :])
        cols = lax.broadcasted_iota(jnp.int32, (RB, S), 1)
        rows = lax.broadcasted_iota(jnp.int32, (RB, S), 0) + j * RB
        mask = (am * (1.0 / H) > 0.1) & (cols != rows)
        cnt_ref[...] = jnp.full((1, 1, 128), jnp.sum(mask.astype(F32)))
        hh = jnp.maximum(_dot(y, w1_ref[...]) + b1_ref[...], 0.0)
        n_ref[...] = _dot(hh, w2_ref[...]) + b2_ref[...]


def _am_body(q_ref, k_ref, am_ref):
    am = jnp.zeros((RB, S), F32)
    for h in range(H):
        sl = slice(h * DH, (h + 1) * DH)
        s = _dot(q_ref[:, sl], k_ref[:, sl], ((1,), (1,))) * 0.125
        m = jnp.max(s, axis=-1, keepdims=True)
        e = jnp.exp(s - m)
        am = am + e / jnp.sum(e, axis=-1, keepdims=True)
    am_ref[...] = am * (1.0 / H)


def _c1c2_body(n_ref, w_ref, c1_ref, c2_ref):
    n = n_ref[...]
    c1_ref[...] = _dot(n, w_ref[:D, :])
    c2_ref[...] = _dot(n, w_ref[D:, :])


def _msg_body(c1t_ref, c2t_ref, neu_ref, am_ref, rb1_ref, w2t_ref, b2_ref,
              wrel_ref, msg_ref, anyv_ref):
    i = pl.program_id(0)
    c2t = c2t_ref[...]
    neu = neu_ref[...]
    vsum = jnp.zeros((1, S), F32)
    for u in range(UB):
        ug = i * UB + u
        ht = jnp.maximum(c1t_ref[0, :, u:u + 1] + c2t + rb1_ref[...], 0.0)
        logits = _dot(w2t_ref[...], ht) + b2_ref[...]
        best = logits[0:1, :]
        pred = jnp.zeros((1, S), jnp.int32)
        for r in range(1, NR):
            cur = logits[r:r + 1, :]
            take = cur > best
            best = jnp.where(take, cur, best)
            pred = jnp.where(take, r, pred)
        colv = lax.broadcasted_iota(jnp.int32, (1, S), 1)
        adj = am_ref[u:u + 1, :] > 0.1
        valid = adj & (colv != ug) & (pred != 0)
        validf = valid.astype(F32)
        oht = (pred == lax.broadcasted_iota(jnp.int32, (NR, 1), 0)
               ).astype(F32) * validf
        agg = _dot(oht, neu)
        msg = jnp.zeros((1, D), F32)
        for r in range(NR):
            msg = msg + _dot(agg[r:r + 1, :], wrel_ref[r * D:(r + 1) * D, :])
        msg_ref[u:u + 1, :] = msg
        vsum = vsum + validf
    anyv_ref[...] = jnp.full((1, 1, 128), jnp.sum(vsum))


def _final_body(x_ref, n_ref, m_ref, w_ref, b_ref, g_ref, bb_ref, f_ref):
    nr = _dot(n_ref[...] + m_ref[...], w_ref[...]) + b_ref[...]
    f_ref[...] = _ln_rows(x_ref[...] + nr, g_ref[...], bb_ref[...])


def kernel(x, ln1_g, ln1_b, Wq, bq, Wk, bk, Wv, bv, Wo, bo, W1, b1, W2, b2,
           rc_W1, rc_b1, rc_W2, rc_b2, Wrel, s2n_W, s2n_b, lnf_g, lnf_b):
    x2 = x[0]
    row = lambda a: a.reshape(1, -1)
    nb = S // RB

    neural2, cnt = pl.pallas_call(
        _block_body,
        grid=(2 * nb,),
        in_specs=[
            pl.BlockSpec((RB, D), lambda i: (lax.rem(i, nb), 0)),
            pl.BlockSpec((1, D), lambda i: (0, 0)),
            pl.BlockSpec((1, D), lambda i: (0, 0)),
            pl.BlockSpec((D, D), lambda i: (0, 0)),
            pl.BlockSpec((1, D), lambda i: (0, 0)),
            pl.BlockSpec((D, D), lambda i: (0, 0)),
            pl.BlockSpec((1, D), lambda i: (0, 0)),
            pl.BlockSpec((D, D), lambda i: (0, 0)),
            pl.BlockSpec((1, D), lambda i: (0, 0)),
            pl.BlockSpec((D, D), lambda i: (0, 0)),
            pl.BlockSpec((1, D), lambda i: (0, 0)),
            pl.BlockSpec((D, DFF), lambda i: (0, 0)),
            pl.BlockSpec((1, DFF), lambda i: (0, 0)),
            pl.BlockSpec((DFF, D), lambda i: (0, 0)),
            pl.BlockSpec((1, D), lambda i: (0, 0)),
        ],
        out_specs=[
            pl.BlockSpec((RB, D), lambda i: (jnp.maximum(i - nb, 0), 0)),
            pl.BlockSpec((1, 1, 128),
                         lambda i: (jnp.maximum(i - nb, 0), 0, 0)),
        ],
        out_shape=[
            jax.ShapeDtypeStruct((S, D), F32),
            jax.ShapeDtypeStruct((nb, 1, 128), F32),
        ],
        scratch_shapes=[pltpu.VMEM((S, D), F32)] * 2,
    )(x2, row(ln1_g), row(ln1_b), Wq, row(bq), Wk, row(bk), Wv, row(bv),
      Wo, row(bo), W1, row(b1), W2, row(b2))

    n_edges = jnp.sum(cnt[:, 0, 0])

    def _easy():
        return neural2

    def _rare():
        q, k, _v = pl.pallas_call(
            _ln_qkv_body,
            grid=(nb,),
            in_specs=[
                pl.BlockSpec((RB, D), lambda i: (i, 0)),
                pl.BlockSpec((1, D), lambda i: (0, 0)),
                pl.BlockSpec((1, D), lambda i: (0, 0)),
                pl.BlockSpec((D, D), lambda i: (0, 0)),
                pl.BlockSpec((1, D), lambda i: (0, 0)),
                pl.BlockSpec((D, D), lambda i: (0, 0)),
                pl.BlockSpec((1, D), lambda i: (0, 0)),
                pl.BlockSpec((D, D), lambda i: (0, 0)),
                pl.BlockSpec((1, D), lambda i: (0, 0)),
            ],
            out_specs=[pl.BlockSpec((RB, D), lambda i: (i, 0))] * 3,
            out_shape=[jax.ShapeDtypeStruct((S, D), F32)] * 3,
        )(x2, row(ln1_g), row(ln1_b), Wq, row(bq), Wk, row(bk),
          Wv, row(bv))

        am = pl.pallas_call(
            _am_body,
            grid=(nb,),
            in_specs=[
                pl.BlockSpec((RB, D), lambda i: (i, 0)),
                pl.BlockSpec((S, D), lambda i: (0, 0)),
            ],
            out_specs=pl.BlockSpec((RB, S), lambda i: (i, 0)),
            out_shape=jax.ShapeDtypeStruct((S, S), F32),
        )(q, k)

        c1, c2 = pl.pallas_call(
            _c1c2_body,
            grid=(nb,),
            in_specs=[
                pl.BlockSpec((RB, D), lambda i: (i, 0)),
                pl.BlockSpec((2 * D, D), lambda i: (0, 0)),
            ],
            out_specs=[pl.BlockSpec((RB, D), lambda i: (i, 0))] * 2,
            out_shape=[jax.ShapeDtypeStruct((S, D), F32)] * 2,
        )(neural2, rc_W1)

        nub = S // UB
        c1t3 = c1.T.reshape(D, nub, UB).transpose(1, 0, 2)
        msgs, anyv = pl.pallas_call(
            _msg_body,
            grid=(nub,),
            in_specs=[
                pl.BlockSpec((1, D, UB), lambda i: (i, 0, 0)),
                pl.BlockSpec((D, S), lambda i: (0, 0)),
                pl.BlockSpec((S, D), lambda i: (0, 0)),
                pl.BlockSpec((UB, S), lambda i: (i, 0)),
                pl.BlockSpec((D, 1), lambda i: (0, 0)),
                pl.BlockSpec((NR, D), lambda i: (0, 0)),
                pl.BlockSpec((NR, 1), lambda i: (0, 0)),
                pl.BlockSpec((NR * D, D), lambda i: (0, 0)),
            ],
            out_specs=[
                pl.BlockSpec((UB, D), lambda i: (i, 0)),
                pl.BlockSpec((1, 1, 128), lambda i: (i, 0, 0)),
            ],
            out_shape=[
                jax.ShapeDtypeStruct((S, D), F32),
                jax.ShapeDtypeStruct((nub, 1, 128), F32),
            ],
        )(c1t3, c2.T, neural2, am, rc_b1.reshape(D, 1), rc_W2.T,
          rc_b2.reshape(NR, 1), Wrel.reshape(NR * D, D))

        has_edge = jnp.sum(anyv[:, 0, 0]) > 0

        full = pl.pallas_call(
            _final_body,
            grid=(nb,),
            in_specs=[
                pl.BlockSpec((RB, D), lambda i: (i, 0)),
                pl.BlockSpec((RB, D), lambda i: (i, 0)),
                pl.BlockSpec((RB, D), lambda i: (i, 0)),
                pl.BlockSpec((D, D), lambda i: (0, 0)),
                pl.BlockSpec((1, D), lambda i: (0, 0)),
                pl.BlockSpec((1, D), lambda i: (0, 0)),
                pl.BlockSpec((1, D), lambda i: (0, 0)),
            ],
            out_specs=pl.BlockSpec((RB, D), lambda i: (i, 0)),
            out_shape=jax.ShapeDtypeStruct((S, D), F32),
        )(x2, neural2, msgs, s2n_W, row(s2n_b), row(lnf_g), row(lnf_b))

        return jnp.where(has_edge, full, neural2)

    out2 = lax.cond(n_edges > 0, _rare, _easy)
    return out2[None]


# speedup vs baseline: 1.3971x; 1.3971x over previous
import jax
import jax.numpy as jnp
from jax import lax
from jax.experimental import pallas as pl
from jax.experimental.pallas import tpu as pltpu

D = 768
S = 2048
H = 12
DH = 64
DFF = 3072
NR = 8
RB = 256
UB = 8
F32 = jnp.float32


def _dot(a, b, dims=None):
    if dims is None:
        return lax.dot(a, b, preferred_element_type=F32)
    return lax.dot_general(a, b, (dims, ((), ())), preferred_element_type=F32)


def _ln_rows(x, g, b):
    m = jnp.mean(x, axis=-1, keepdims=True)
    xc = x - m
    v = jnp.mean(xc * xc, axis=-1, keepdims=True)
    return xc * lax.rsqrt(v + 1e-5) * g + b


def _ln_qkv_body(x_ref, g_ref, b_ref, wq_ref, bq_ref, wk_ref, bk_ref,
                 wv_ref, bv_ref, q_ref, k_ref, v_ref):
    xn = _ln_rows(x_ref[...], g_ref[...], b_ref[...])
    q_ref[...] = _dot(xn, wq_ref[...]) + bq_ref[...]
    k_ref[...] = _dot(xn, wk_ref[...]) + bk_ref[...]
    v_ref[...] = _dot(xn, wv_ref[...]) + bv_ref[...]


def _attn_ffn_body(q_ref, k_ref, v_ref, wo_ref, bo_ref, w1_ref, b1_ref,
                   w2_ref, b2_ref, n_ref, cnt_ref):
    i = pl.program_id(0)
    am = jnp.zeros((RB, S), F32)
    o = []
    for h in range(H):
        sl = slice(h * DH, (h + 1) * DH)
        s = _dot(q_ref[:, sl], k_ref[:, sl], ((1,), (1,))) * 0.125
        m = jnp.max(s, axis=-1, keepdims=True)
        e = jnp.exp(s - m)
        p = e / jnp.sum(e, axis=-1, keepdims=True)
        am = am + p
        o.append(_dot(p, v_ref[:, sl]))
    cols = lax.broadcasted_iota(jnp.int32, (RB, S), 1)
    rows = lax.broadcasted_iota(jnp.int32, (RB, S), 0) + i * RB
    mask = (am * (1.0 / H) > 0.1) & (cols != rows)
    cnt_ref[...] = jnp.full((1, 1, 128), jnp.sum(mask.astype(F32)))
    y = _dot(jnp.concatenate(o, axis=1), wo_ref[...]) + bo_ref[...]
    hh = jnp.maximum(_dot(y, w1_ref[...]) + b1_ref[...], 0.0)
    n_ref[...] = _dot(hh, w2_ref[...]) + b2_ref[...]


def _am_body(q_ref, k_ref, am_ref):
    am = jnp.zeros((RB, S), F32)
    for h in range(H):
        sl = slice(h * DH, (h + 1) * DH)
        s = _dot(q_ref[:, sl], k_ref[:, sl], ((1,), (1,))) * 0.125
        m = jnp.max(s, axis=-1, keepdims=True)
        e = jnp.exp(s - m)
        am = am + e / jnp.sum(e, axis=-1, keepdims=True)
    am_ref[...] = am * (1.0 / H)


def _c1c2_body(n_ref, w_ref, c1_ref, c2_ref):
    n = n_ref[...]
    c1_ref[...] = _dot(n, w_ref[:D, :])
    c2_ref[...] = _dot(n, w_ref[D:, :])


def _msg_body(c1t_ref, c2t_ref, neu_ref, am_ref, rb1_ref, w2t_ref, b2_ref,
              wrel_ref, msg_ref, anyv_ref):
    i = pl.program_id(0)
    c2t = c2t_ref[...]
    neu = neu_ref[...]
    vsum = jnp.zeros((1, S), F32)
    for u in range(UB):
        ug = i * UB + u
        ht = jnp.maximum(c1t_ref[0, :, u:u + 1] + c2t + rb1_ref[...], 0.0)
        logits = _dot(w2t_ref[...], ht) + b2_ref[...]
        best = logits[0:1, :]
        pred = jnp.zeros((1, S), jnp.int32)
        for r in range(1, NR):
            cur = logits[r:r + 1, :]
            take = cur > best
            best = jnp.where(take, cur, best)
            pred = jnp.where(take, r, pred)
        colv = lax.broadcasted_iota(jnp.int32, (1, S), 1)
        adj = am_ref[u:u + 1, :] > 0.1
        valid = adj & (colv != ug) & (pred != 0)
        validf = valid.astype(F32)
        oht = (pred == lax.broadcasted_iota(jnp.int32, (NR, 1), 0)
               ).astype(F32) * validf
        agg = _dot(oht, neu)
        msg = jnp.zeros((1, D), F32)
        for r in range(NR):
            msg = msg + _dot(agg[r:r + 1, :], wrel_ref[r * D:(r + 1) * D, :])
        msg_ref[u:u + 1, :] = msg
        vsum = vsum + validf
    anyv_ref[...] = jnp.full((1, 1, 128), jnp.sum(vsum))


def _final_body(x_ref, n_ref, m_ref, w_ref, b_ref, g_ref, bb_ref, f_ref):
    nr = _dot(n_ref[...] + m_ref[...], w_ref[...]) + b_ref[...]
    f_ref[...] = _ln_rows(x_ref[...] + nr, g_ref[...], bb_ref[...])


def kernel(x, ln1_g, ln1_b, Wq, bq, Wk, bk, Wv, bv, Wo, bo, W1, b1, W2, b2,
           rc_W1, rc_b1, rc_W2, rc_b2, Wrel, s2n_W, s2n_b, lnf_g, lnf_b):
    x2 = x[0]
    row = lambda a: a.reshape(1, -1)
    nb = S // RB

    q, k, v = pl.pallas_call(
        _ln_qkv_body,
        grid=(nb,),
        in_specs=[
            pl.BlockSpec((RB, D), lambda i: (i, 0)),
            pl.BlockSpec((1, D), lambda i: (0, 0)),
            pl.BlockSpec((1, D), lambda i: (0, 0)),
            pl.BlockSpec((D, D), lambda i: (0, 0)),
            pl.BlockSpec((1, D), lambda i: (0, 0)),
            pl.BlockSpec((D, D), lambda i: (0, 0)),
            pl.BlockSpec((1, D), lambda i: (0, 0)),
            pl.BlockSpec((D, D), lambda i: (0, 0)),
            pl.BlockSpec((1, D), lambda i: (0, 0)),
        ],
        out_specs=[pl.BlockSpec((RB, D), lambda i: (i, 0))] * 3,
        out_shape=[jax.ShapeDtypeStruct((S, D), F32)] * 3,
    )(x2, row(ln1_g), row(ln1_b), Wq, row(bq), Wk, row(bk), Wv, row(bv))

    neural2, cnt = pl.pallas_call(
        _attn_ffn_body,
        grid=(nb,),
        in_specs=[
            pl.BlockSpec((RB, D), lambda i: (i, 0)),
            pl.BlockSpec((S, D), lambda i: (0, 0)),
            pl.BlockSpec((S, D), lambda i: (0, 0)),
            pl.BlockSpec((D, D), lambda i: (0, 0)),
            pl.BlockSpec((1, D), lambda i: (0, 0)),
            pl.BlockSpec((D, DFF), lambda i: (0, 0)),
            pl.BlockSpec((1, DFF), lambda i: (0, 0)),
            pl.BlockSpec((DFF, D), lambda i: (0, 0)),
            pl.BlockSpec((1, D), lambda i: (0, 0)),
        ],
        out_specs=[
            pl.BlockSpec((RB, D), lambda i: (i, 0)),
            pl.BlockSpec((1, 1, 128), lambda i: (i, 0, 0)),
        ],
        out_shape=[
            jax.ShapeDtypeStruct((S, D), F32),
            jax.ShapeDtypeStruct((nb, 1, 128), F32),
        ],
    )(q, k, v, Wo, row(bo), W1, row(b1), W2, row(b2))

    n_edges = jnp.sum(cnt[:, 0, 0])

    def _easy():
        return neural2

    def _rare():
        am = pl.pallas_call(
            _am_body,
            grid=(nb,),
            in_specs=[
                pl.BlockSpec((RB, D), lambda i: (i, 0)),
                pl.BlockSpec((S, D), lambda i: (0, 0)),
            ],
            out_specs=pl.BlockSpec((RB, S), lambda i: (i, 0)),
            out_shape=jax.ShapeDtypeStruct((S, S), F32),
        )(q, k)

        c1, c2 = pl.pallas_call(
            _c1c2_body,
            grid=(nb,),
            in_specs=[
                pl.BlockSpec((RB, D), lambda i: (i, 0)),
                pl.BlockSpec((2 * D, D), lambda i: (0, 0)),
            ],
            out_specs=[pl.BlockSpec((RB, D), lambda i: (i, 0))] * 2,
            out_shape=[jax.ShapeDtypeStruct((S, D), F32)] * 2,
        )(neural2, rc_W1)

        nub = S // UB
        c1t3 = c1.T.reshape(D, nub, UB).transpose(1, 0, 2)
        msgs, anyv = pl.pallas_call(
            _msg_body,
            grid=(nub,),
            in_specs=[
                pl.BlockSpec((1, D, UB), lambda i: (i, 0, 0)),
                pl.BlockSpec((D, S), lambda i: (0, 0)),
                pl.BlockSpec((S, D), lambda i: (0, 0)),
                pl.BlockSpec((UB, S), lambda i: (i, 0)),
                pl.BlockSpec((D, 1), lambda i: (0, 0)),
                pl.BlockSpec((NR, D), lambda i: (0, 0)),
                pl.BlockSpec((NR, 1), lambda i: (0, 0)),
                pl.BlockSpec((NR * D, D), lambda i: (0, 0)),
            ],
            out_specs=[
                pl.BlockSpec((UB, D), lambda i: (i, 0)),
                pl.BlockSpec((1, 1, 128), lambda i: (i, 0, 0)),
            ],
            out_shape=[
                jax.ShapeDtypeStruct((S, D), F32),
                jax.ShapeDtypeStruct((nub, 1, 128), F32),
            ],
        )(c1t3, c2.T, neural2, am, rc_b1.reshape(D, 1), rc_W2.T,
          rc_b2.reshape(NR, 1), Wrel.reshape(NR * D, D))

        has_edge = jnp.sum(anyv[:, 0, 0]) > 0

        full = pl.pallas_call(
            _final_body,
            grid=(nb,),
            in_specs=[
                pl.BlockSpec((RB, D), lambda i: (i, 0)),
                pl.BlockSpec((RB, D), lambda i: (i, 0)),
                pl.BlockSpec((RB, D), lambda i: (i, 0)),
                pl.BlockSpec((D, D), lambda i: (0, 0)),
                pl.BlockSpec((1, D), lambda i: (0, 0)),
                pl.BlockSpec((1, D), lambda i: (0, 0)),
                pl.BlockSpec((1, D), lambda i: (0, 0)),
            ],
            out_specs=pl.BlockSpec((RB, D), lambda i: (i, 0)),
            out_shape=jax.ShapeDtypeStruct((S, D), F32),
        )(x2, neural2, msgs, s2n_W, row(s2n_b), row(lnf_g), row(lnf_b))

        return jnp.where(has_edge, full, neural2)

    out2 = lax.cond(n_edges > 0, _rare, _easy)
    return out2[None]
